# baseline (speedup 1.0000x reference)
import functools

import jax
import jax.numpy as jnp
from jax import lax
from jax.experimental import pallas as pl
from jax.experimental.pallas import tpu as pltpu
from jax.experimental.pallas import tpu_sc as plsc

S, D = 2048, 768
E, K, H = 8, 2, 1536
P = K * S
BK = 256
NB = P // BK + (E - 1)
NBK = NB * BK
NBKD = NBK
NBPAD = 24
NC, NS = 2, 16
NW = NC * NS
RPT = NBKD // NW
GCH = ((0, 96), (96, 88))
GB = 96
DW = D // 2
TPB = S // NW
CH = 512


def _router_kernel(x_ref, wg_ref, bg_ref, gate_ref, pos_ref, qpair_ref,
                   beid_ref, oh_ref):
    xv = x_ref[0]
    logits = jnp.dot(xv, wg_ref[...], preferred_element_type=jnp.float32)
    logits = logits + bg_ref[...]
    m = jnp.max(logits, axis=1, keepdims=True)
    ex = jnp.exp(logits - m)
    probs = ex / jnp.sum(ex, axis=1, keepdims=True)
    gate_ref[...] = probs

    idx = lax.broadcasted_iota(jnp.int32, (S, E), 1)
    p1 = jnp.max(probs, axis=1, keepdims=True)
    i1 = jnp.min(jnp.where(probs == p1, idx, E), axis=1, keepdims=True)
    pm = jnp.where(idx == i1, -jnp.inf, probs)
    p2 = jnp.max(pm, axis=1, keepdims=True)
    i2 = jnp.min(jnp.where(pm == p2, idx, E), axis=1, keepdims=True)
    t = jnp.exp(p2 - p1)
    q1 = 1.0 / (1.0 + t)
    q2 = t / (1.0 + t)

    i12 = jnp.concatenate([i1, i2], axis=0)
    qpair_ref[...] = jnp.concatenate([q1, q2], axis=0)
    lane8 = lax.broadcasted_iota(jnp.int32, (P, E), 1)
    oh = (i12 == lane8).astype(jnp.float32)
    oh_ref[...] = oh

    cnt = jnp.sum(oh, axis=0, keepdims=True)
    npad = jnp.ceil(cnt / BK) * BK
    r8 = lax.broadcasted_iota(jnp.int32, (E, E), 0)
    c8 = lax.broadcasted_iota(jnp.int32, (E, E), 1)
    excl8 = (r8 < c8).astype(jnp.float32)
    poff = jnp.dot(npad, excl8, preferred_element_type=jnp.float32)

    rr = lax.broadcasted_iota(jnp.int32, (CH, CH), 0)
    cc = lax.broadcasted_iota(jnp.int32, (CH, CH), 1)
    ltri = (rr > cc).astype(jnp.float32)

    def body(i, base):
        blk = oh_ref[pl.ds(i * CH, CH), :]
        rank = jnp.dot(ltri, blk, preferred_element_type=jnp.float32)
        row = jnp.sum((rank + base + poff) * blk, axis=1, keepdims=True)
        pos_ref[pl.ds(i * CH, CH), :] = row.astype(jnp.int32)
        return base + jnp.sum(blk, axis=0, keepdims=True)

    lax.fori_loop(0, P // CH, body, jnp.zeros((1, E), jnp.float32))

    nblk = npad / BK
    blkoff = jnp.dot(nblk, excl8, preferred_element_type=jnp.float32)
    biota = lax.broadcasted_iota(jnp.int32, (NBPAD, E), 0).astype(jnp.float32)
    ge = (biota >= blkoff).astype(jnp.float32)
    beid_ref[...] = (jnp.sum(ge, axis=1, keepdims=True) - 1.0).astype(jnp.int32)


PPT = P // NS
ZPT = NBKD // NS
ZPT16 = ZPT


def _sc_dispatch(pos_hbm, q_hbm, x_hbm, xg_hbm, wr_hbm,
                 myidx, tokv, qv, zbi, zbf, idx_v, buf_a, buf_b, rt_sh, wr_sh,
                 sga, sgb, swa, swb):
    bufs = (buf_a, buf_b)
    gsems = (sga, sgb)
    wsems = (swa, swb)
    cid = lax.axis_index("c")
    sid = lax.axis_index("s")
    pb = sid * PPT

    with jax.named_scope("disp_load"):
        for j in range(2):
            pltpu.sync_copy(pos_hbm.at[pl.ds(pb + j * 128, 128)], myidx.at[j])
            pltpu.sync_copy(q_hbm.at[pl.ds(pb + j * 128, 128)], qv.at[j])

    def tbody(i, _):
        for j in range(2):
            sl = pl.ds(i * 16, 16)
            tok = (lax.iota(jnp.int32, 16) + (pb + j * 128 + i * 16)) & (S - 1)
            tokv[j, sl] = tok - (myidx[j, sl] & (S - 1))
        return 0

    lax.fori_loop(0, 8, tbody, 0)

    zf = jnp.zeros((16,), jnp.float32)
    zb = sid * ZPT

    def zbody(i, _):
        zbi[pl.ds(i * 16, 16)] = (lax.iota(jnp.int32, 16) + (zb + i * 16)) & (S - 1)
        zbf[pl.ds(i * 16, 16)] = zf
        return 0

    with jax.named_scope("disp_zero"):
        lax.fori_loop(0, ZPT16 // 16, zbody, 0)
        pltpu.sync_copy(zbi.at[pl.ds(0, ZPT)], rt_sh.at[pl.ds(zb, ZPT)])
        pltpu.sync_copy(zbf.at[pl.ds(0, ZPT)], wr_sh.at[pl.ds(zb, ZPT)])
        plsc.subcore_barrier()
    with jax.named_scope("disp_scatter"):
        for j in range(2):
            pltpu.sync_copy(tokv.at[j], rt_sh.at[myidx.at[j]], add=True)
            pltpu.sync_copy(qv.at[j], wr_sh.at[myidx.at[j]], add=True)
        plsc.subcore_barrier()

    @pl.when(cid == 0)
    def _():
        pltpu.sync_copy(wr_sh.at[pl.ds(zb, ZPT)], zbf.at[pl.ds(0, ZPT)])
        pltpu.sync_copy(zbf.at[pl.ds(0, ZPT)], wr_hbm.at[pl.ds(zb, ZPT)])

    with jax.named_scope("disp_gather"):
        tid = cid * NS + sid
        base = tid * RPT
        pltpu.sync_copy(rt_sh.at[pl.ds(base, RPT)], idx_v)

        def gather(ci):
            off, n = GCH[ci]
            return pltpu.async_copy(x_hbm.at[idx_v.at[pl.ds(off, n)]],
                                    bufs[ci].at[pl.ds(0, n)], gsems[ci])

        def write(ci, sem):
            off, n = GCH[ci]
            return pltpu.async_copy(bufs[ci].at[pl.ds(0, n)],
                                    xg_hbm.at[pl.ds(base + off, n)], sem)

        g0 = gather(0)
        g1 = gather(1)
        g0.wait()
        w0 = write(0, wsems[0])
        g1.wait()
        w1 = write(1, wsems[1])
        w0.wait()
        w1.wait()


def _gmm_kernel(beid_ref, xg_ref, w1_ref, b1_ref, w2_ref, b2_ref, wr_ref,
                y_ref):
    del beid_ref
    xv = xg_ref[...]
    acc = jnp.zeros((BK, D), jnp.float32)
    for hc in range(2):
        sl = slice(hc * (H // 2), (hc + 1) * (H // 2))
        h = jnp.maximum(
            jnp.dot(xv, w1_ref[0, :, sl].astype(jnp.bfloat16),
                    preferred_element_type=jnp.float32)
            + b1_ref[0, :, sl], 0.0)
        acc = acc + jnp.dot(h.astype(jnp.bfloat16),
                            w2_ref[0, sl, :].astype(jnp.bfloat16),
                            preferred_element_type=jnp.float32)
    y_ref[...] = (acc + b2_ref[0]) * wr_ref[...]


def _sc_combine(pos_hbm, y_hbm, out_hbm, idx1_v, idx2_v, buf1, buf2, sem):
    cid = lax.axis_index("c")
    sid = lax.axis_index("s")
    tid = cid * NS + sid
    tb = tid * TPB
    pltpu.sync_copy(pos_hbm.at[pl.ds(tb, TPB)], idx1_v)
    pltpu.sync_copy(pos_hbm.at[pl.ds(S + tb, TPB)], idx2_v)
    c1 = pltpu.async_copy(y_hbm.at[idx1_v], buf1, sem)
    c2 = pltpu.async_copy(y_hbm.at[idx2_v], buf2, sem)
    c1.wait()
    c2.wait()

    def rbody(r, _):
        for c in range(D // 16):
            sl = pl.ds(c * 16, 16)
            buf1[r, sl] = buf1[r, sl] + buf2[r, sl]
        return 0

    lax.fori_loop(0, TPB, rbody, 0)
    pltpu.sync_copy(buf1, out_hbm.at[0, pl.ds(tb, TPB)])


@functools.lru_cache(maxsize=None)
def _sc_calls():
    mesh = plsc.VectorSubcoreMesh(core_axis_name="c", subcore_axis_name="s",
                                  num_cores=NC, num_subcores=NS)
    dispatch = pl.kernel(
        _sc_dispatch,
        out_type=[jax.ShapeDtypeStruct((NBKD, DW), jnp.int32),
                  jax.ShapeDtypeStruct((NBKD,), jnp.float32)],
        mesh=mesh,
        scratch_types=[
            pltpu.VMEM((2, 128), jnp.int32),
            pltpu.VMEM((2, 128), jnp.int32),
            pltpu.VMEM((2, 128), jnp.float32),
            pltpu.VMEM((ZPT16,), jnp.int32),
            pltpu.VMEM((ZPT16,), jnp.float32),
            pltpu.VMEM((RPT,), jnp.int32),
            pltpu.VMEM((GB, DW), jnp.int32),
            pltpu.VMEM((GB, DW), jnp.int32),
            pltpu.VMEM_SHARED((NBKD,), jnp.int32),
            pltpu.VMEM_SHARED((NBKD,), jnp.float32),
            pltpu.SemaphoreType.DMA,
            pltpu.SemaphoreType.DMA,
            pltpu.SemaphoreType.DMA,
            pltpu.SemaphoreType.DMA,
        ],
        compiler_params=pltpu.CompilerParams(needs_layout_passes=False),
    )
    combine = pl.kernel(
        _sc_combine,
        out_type=jax.ShapeDtypeStruct((1, S, D), jnp.float32),
        mesh=mesh,
        scratch_types=[
            pltpu.VMEM((TPB,), jnp.int32),
            pltpu.VMEM((TPB,), jnp.int32),
            pltpu.VMEM((TPB, D), jnp.float32),
            pltpu.VMEM((TPB, D), jnp.float32),
            pltpu.SemaphoreType.DMA,
        ],
        compiler_params=pltpu.CompilerParams(needs_layout_passes=False),
    )
    return dispatch, combine


@jax.jit
def kernel(x, Wg, bg, W1, b1, W2, b2):
    B = x.shape[0]

    gate, pos, qpair, beid = pl.pallas_call(
        _router_kernel,
        out_shape=[
            jax.ShapeDtypeStruct((S, E), jnp.float32),
            jax.ShapeDtypeStruct((P, 1), jnp.int32),
            jax.ShapeDtypeStruct((P, 1), jnp.float32),
            jax.ShapeDtypeStruct((NBPAD, 1), jnp.int32),
        ],
        scratch_shapes=[pltpu.VMEM((P, E), jnp.float32)],
    )(x.reshape(1, S, D), Wg, bg.reshape(1, E))

    xb = jax.lax.bitcast_convert_type(
        x.reshape(S, D).astype(jnp.bfloat16).reshape(S, DW, 2), jnp.int32)

    dispatch_call, combine_call = _sc_calls()
    pos1 = pos.reshape(P)
    xg, wrow = dispatch_call(pos1, qpair.reshape(P), xb)
    xgb = jax.lax.bitcast_convert_type(xg, jnp.bfloat16).reshape(NBKD, D)

    y = pl.pallas_call(
        _gmm_kernel,
        grid_spec=pltpu.PrefetchScalarGridSpec(
            num_scalar_prefetch=1,
            grid=(NB,),
            in_specs=[
                pl.BlockSpec((BK, D), lambda i, beid: (i, 0)),
                pl.BlockSpec((1, D, H), lambda i, beid: (beid[i], 0, 0)),
                pl.BlockSpec((1, 1, H), lambda i, beid: (beid[i], 0, 0)),
                pl.BlockSpec((1, H, D), lambda i, beid: (beid[i], 0, 0)),
                pl.BlockSpec((1, 1, D), lambda i, beid: (beid[i], 0, 0)),
                pl.BlockSpec((BK, 1), lambda i, beid: (i, 0)),
            ],
            out_specs=pl.BlockSpec((BK, D), lambda i, beid: (i, 0)),
        ),
        out_shape=jax.ShapeDtypeStruct((NBK, D), jnp.float32),
    )(beid.reshape(NBPAD), xgb, W1, b1.reshape(E, 1, H), W2,
      b2.reshape(E, 1, D), wrow.reshape(NBKD, 1))

    out = combine_call(pos1, y)
    return out.reshape(B, S, D), gate.reshape(B, S, E)


# speedup vs baseline: 1.9329x; 1.9329x over previous
import functools

import jax
import jax.numpy as jnp
from jax import lax
from jax.experimental import pallas as pl
from jax.experimental.pallas import tpu as pltpu
from jax.experimental.pallas import tpu_sc as plsc

S, D = 2048, 768
E, K, H = 8, 2, 1536
P = K * S
BK = 256
NB = P // BK + (E - 1)
NBK = NB * BK
NBKD = NBK
NBPAD = 24
NC, NS = 2, 16
NW = NC * NS
RPT = NBKD // NW
GCH = ((0, 64), (64, 64), (128, 56))
GB = 64
TPB = S // NW
CH = 512


def _router_kernel(x_ref, wg_ref, bg_ref, gate_ref, pos_ref, qpair_ref,
                   beid_ref, oh_ref):
    xv = x_ref[0]
    logits = jnp.dot(xv, wg_ref[...], preferred_element_type=jnp.float32)
    logits = logits + bg_ref[...]
    m = jnp.max(logits, axis=1, keepdims=True)
    ex = jnp.exp(logits - m)
    probs = ex / jnp.sum(ex, axis=1, keepdims=True)
    gate_ref[...] = probs

    idx = lax.broadcasted_iota(jnp.int32, (S, E), 1)
    p1 = jnp.max(probs, axis=1, keepdims=True)
    i1 = jnp.min(jnp.where(probs == p1, idx, E), axis=1, keepdims=True)
    pm = jnp.where(idx == i1, -jnp.inf, probs)
    p2 = jnp.max(pm, axis=1, keepdims=True)
    i2 = jnp.min(jnp.where(pm == p2, idx, E), axis=1, keepdims=True)
    t = jnp.exp(p2 - p1)
    q1 = 1.0 / (1.0 + t)
    q2 = t / (1.0 + t)

    i12 = jnp.concatenate([i1, i2], axis=0)
    qpair_ref[...] = jnp.concatenate([q1, q2], axis=0)
    lane8 = lax.broadcasted_iota(jnp.int32, (P, E), 1)
    oh = (i12 == lane8).astype(jnp.float32)
    oh_ref[...] = oh

    cnt = jnp.sum(oh, axis=0, keepdims=True)
    npad = jnp.ceil(cnt / BK) * BK
    r8 = lax.broadcasted_iota(jnp.int32, (E, E), 0)
    c8 = lax.broadcasted_iota(jnp.int32, (E, E), 1)
    excl8 = (r8 < c8).astype(jnp.float32)
    poff = jnp.dot(npad, excl8, preferred_element_type=jnp.float32)

    rr = lax.broadcasted_iota(jnp.int32, (CH, CH), 0)
    cc = lax.broadcasted_iota(jnp.int32, (CH, CH), 1)
    ltri = (rr > cc).astype(jnp.float32)

    def body(i, base):
        blk = oh_ref[pl.ds(i * CH, CH), :]
        rank = jnp.dot(ltri, blk, preferred_element_type=jnp.float32)
        row = jnp.sum((rank + base + poff) * blk, axis=1, keepdims=True)
        pos_ref[pl.ds(i * CH, CH), :] = row.astype(jnp.int32)
        return base + jnp.sum(blk, axis=0, keepdims=True)

    lax.fori_loop(0, P // CH, body, jnp.zeros((1, E), jnp.float32))

    nblk = npad / BK
    blkoff = jnp.dot(nblk, excl8, preferred_element_type=jnp.float32)
    biota = lax.broadcasted_iota(jnp.int32, (NBPAD, E), 0).astype(jnp.float32)
    ge = (biota >= blkoff).astype(jnp.float32)
    beid_ref[...] = (jnp.sum(ge, axis=1, keepdims=True) - 1.0).astype(jnp.int32)


PPT = P // NS
ZPT = NBKD // NS
ZPT16 = ZPT


def _sc_dispatch(pos_hbm, q_hbm, x_hbm, xg_hbm, wr_hbm,
                 myidx, tokv, qv, zbi, zbf, idx_v, buf_a, buf_b, rt_sh, wr_sh,
                 sga, sgb, swa, swb):
    bufs = (buf_a, buf_b)
    gsems = (sga, sgb)
    wsems = (swa, swb)
    cid = lax.axis_index("c")
    sid = lax.axis_index("s")
    pb = sid * PPT

    with jax.named_scope("disp_load"):
        for j in range(2):
            pltpu.sync_copy(pos_hbm.at[pl.ds(pb + j * 128, 128)], myidx.at[j])
            pltpu.sync_copy(q_hbm.at[pl.ds(pb + j * 128, 128)], qv.at[j])

    def tbody(i, _):
        for j in range(2):
            sl = pl.ds(i * 16, 16)
            tok = (lax.iota(jnp.int32, 16) + (pb + j * 128 + i * 16)) & (S - 1)
            tokv[j, sl] = tok - (myidx[j, sl] & (S - 1))
        return 0

    lax.fori_loop(0, 8, tbody, 0)

    zf = jnp.zeros((16,), jnp.float32)
    zb = sid * ZPT

    def zbody(i, _):
        zbi[pl.ds(i * 16, 16)] = (lax.iota(jnp.int32, 16) + (zb + i * 16)) & (S - 1)
        zbf[pl.ds(i * 16, 16)] = zf
        return 0

    with jax.named_scope("disp_zero"):
        lax.fori_loop(0, ZPT16 // 16, zbody, 0)
        pltpu.sync_copy(zbi.at[pl.ds(0, ZPT)], rt_sh.at[pl.ds(zb, ZPT)])
        pltpu.sync_copy(zbf.at[pl.ds(0, ZPT)], wr_sh.at[pl.ds(zb, ZPT)])
        plsc.subcore_barrier()
    with jax.named_scope("disp_scatter"):
        for j in range(2):
            pltpu.sync_copy(tokv.at[j], rt_sh.at[myidx.at[j]], add=True)
            pltpu.sync_copy(qv.at[j], wr_sh.at[myidx.at[j]], add=True)
        plsc.subcore_barrier()

    @pl.when(cid == 0)
    def _():
        pltpu.sync_copy(wr_sh.at[pl.ds(zb, ZPT)], zbf.at[pl.ds(0, ZPT)])
        pltpu.sync_copy(zbf.at[pl.ds(0, ZPT)], wr_hbm.at[pl.ds(zb, ZPT)])

    with jax.named_scope("disp_gather"):
        tid = cid * NS + sid
        base = tid * RPT
        pltpu.sync_copy(rt_sh.at[pl.ds(base, RPT)], idx_v)

        def gather(ci):
            off, n = GCH[ci]
            return pltpu.async_copy(x_hbm.at[idx_v.at[pl.ds(off, n)]],
                                    bufs[ci % 2].at[pl.ds(0, n)], gsems[ci % 2])

        def write(ci, sem):
            off, n = GCH[ci]
            return pltpu.async_copy(bufs[ci % 2].at[pl.ds(0, n)],
                                    xg_hbm.at[pl.ds(base + off, n)], sem)

        g0 = gather(0)
        g1 = gather(1)
        g0.wait()
        w0 = write(0, wsems[0])
        w0.wait()
        g2 = gather(2)
        g1.wait()
        w1 = write(1, wsems[1])
        g2.wait()
        w2 = write(2, wsems[0])
        w1.wait()
        w2.wait()


def _gmm_kernel(beid_ref, xg_ref, w1_ref, b1_ref, w2_ref, b2_ref, wr_ref,
                y_ref):
    del beid_ref
    xv = xg_ref[...].astype(jnp.bfloat16)
    acc = jnp.zeros((BK, D), jnp.float32)
    for hc in range(2):
        sl = slice(hc * (H // 2), (hc + 1) * (H // 2))
        h = jnp.maximum(
            jnp.dot(xv, w1_ref[0, :, sl].astype(jnp.bfloat16),
                    preferred_element_type=jnp.float32)
            + b1_ref[0, :, sl], 0.0)
        acc = acc + jnp.dot(h.astype(jnp.bfloat16),
                            w2_ref[0, sl, :].astype(jnp.bfloat16),
                            preferred_element_type=jnp.float32)
    y_ref[...] = (acc + b2_ref[0]) * wr_ref[...]


def _sc_combine(pos_hbm, y_hbm, out_hbm, idx1_v, idx2_v, buf1, buf2, sem):
    cid = lax.axis_index("c")
    sid = lax.axis_index("s")
    tid = cid * NS + sid
    tb = tid * TPB
    pltpu.sync_copy(pos_hbm.at[pl.ds(tb, TPB)], idx1_v)
    pltpu.sync_copy(pos_hbm.at[pl.ds(S + tb, TPB)], idx2_v)
    c1 = pltpu.async_copy(y_hbm.at[idx1_v], buf1, sem)
    c2 = pltpu.async_copy(y_hbm.at[idx2_v], buf2, sem)
    c1.wait()
    c2.wait()

    def rbody(r, _):
        for c in range(D // 16):
            sl = pl.ds(c * 16, 16)
            buf1[r, sl] = buf1[r, sl] + buf2[r, sl]
        return 0

    lax.fori_loop(0, TPB, rbody, 0)
    pltpu.sync_copy(buf1, out_hbm.at[0, pl.ds(tb, TPB)])


@functools.lru_cache(maxsize=None)
def _sc_calls():
    mesh = plsc.VectorSubcoreMesh(core_axis_name="c", subcore_axis_name="s",
                                  num_cores=NC, num_subcores=NS)
    dispatch = pl.kernel(
        _sc_dispatch,
        out_type=[jax.ShapeDtypeStruct((NBKD, D), jnp.float32),
                  jax.ShapeDtypeStruct((NBKD,), jnp.float32)],
        mesh=mesh,
        scratch_types=[
            pltpu.VMEM((2, 128), jnp.int32),
            pltpu.VMEM((2, 128), jnp.int32),
            pltpu.VMEM((2, 128), jnp.float32),
            pltpu.VMEM((ZPT16,), jnp.int32),
            pltpu.VMEM((ZPT16,), jnp.float32),
            pltpu.VMEM((RPT,), jnp.int32),
            pltpu.VMEM((GB, D), jnp.float32),
            pltpu.VMEM((GB, D), jnp.float32),
            pltpu.VMEM_SHARED((NBKD,), jnp.int32),
            pltpu.VMEM_SHARED((NBKD,), jnp.float32),
            pltpu.SemaphoreType.DMA,
            pltpu.SemaphoreType.DMA,
            pltpu.SemaphoreType.DMA,
            pltpu.SemaphoreType.DMA,
        ],
        compiler_params=pltpu.CompilerParams(needs_layout_passes=False),
    )
    combine = pl.kernel(
        _sc_combine,
        out_type=jax.ShapeDtypeStruct((1, S, D), jnp.float32),
        mesh=mesh,
        scratch_types=[
            pltpu.VMEM((TPB,), jnp.int32),
            pltpu.VMEM((TPB,), jnp.int32),
            pltpu.VMEM((TPB, D), jnp.float32),
            pltpu.VMEM((TPB, D), jnp.float32),
            pltpu.SemaphoreType.DMA,
        ],
        compiler_params=pltpu.CompilerParams(needs_layout_passes=False),
    )
    return dispatch, combine


@jax.jit
def kernel(x, Wg, bg, W1, b1, W2, b2):
    B = x.shape[0]

    gate, pos, qpair, beid = pl.pallas_call(
        _router_kernel,
        out_shape=[
            jax.ShapeDtypeStruct((S, E), jnp.float32),
            jax.ShapeDtypeStruct((P, 1), jnp.int32),
            jax.ShapeDtypeStruct((P, 1), jnp.float32),
            jax.ShapeDtypeStruct((NBPAD, 1), jnp.int32),
        ],
        scratch_shapes=[pltpu.VMEM((P, E), jnp.float32)],
    )(x.reshape(1, S, D), Wg, bg.reshape(1, E))

    dispatch_call, combine_call = _sc_calls()
    pos1 = pos.reshape(P)
    xg, wrow = dispatch_call(pos1, qpair.reshape(P), x.reshape(S, D))

    y = pl.pallas_call(
        _gmm_kernel,
        grid_spec=pltpu.PrefetchScalarGridSpec(
            num_scalar_prefetch=1,
            grid=(NB,),
            in_specs=[
                pl.BlockSpec((BK, D), lambda i, beid: (i, 0)),
                pl.BlockSpec((1, D, H), lambda i, beid: (beid[i], 0, 0)),
                pl.BlockSpec((1, 1, H), lambda i, beid: (beid[i], 0, 0)),
                pl.BlockSpec((1, H, D), lambda i, beid: (beid[i], 0, 0)),
                pl.BlockSpec((1, 1, D), lambda i, beid: (beid[i], 0, 0)),
                pl.BlockSpec((BK, 1), lambda i, beid: (i, 0)),
            ],
            out_specs=pl.BlockSpec((BK, D), lambda i, beid: (i, 0)),
        ),
        out_shape=jax.ShapeDtypeStruct((NBK, D), jnp.float32),
    )(beid.reshape(NBPAD), xg, W1, b1.reshape(E, 1, H), W2,
      b2.reshape(E, 1, D), wrow.reshape(NBKD, 1))

    out = combine_call(pos1, y)
    return out.reshape(B, S, D), gate.reshape(B, S, E)
